# baseline (speedup 1.0000x reference)
import functools

import jax
import jax.numpy as jnp
import numpy as np
from jax.experimental import pallas as pl
from jax.experimental.pallas import tpu as pltpu

_K = 32
_BLK = 256
_BS_ITERS = 31


def _rotl32(x, d):
    x = x.astype(np.uint32)
    return ((x << np.uint32(d)) | (x >> np.uint32(32 - d))).astype(np.uint32)


def _threefry2x32_np(k0, k1, x0, x1):
    r0 = (13, 15, 26, 6)
    r1 = (17, 29, 16, 24)
    keys = (np.uint32(k0), np.uint32(k1),
            np.uint32(np.uint32(k0) ^ np.uint32(k1) ^ np.uint32(0x1BD11BDA)))
    x0 = (x0 + keys[0]).astype(np.uint32)
    x1 = (x1 + keys[1]).astype(np.uint32)
    for r in range(5):
        for d in r0 if r % 2 == 0 else r1:
            x0 = (x0 + x1).astype(np.uint32)
            x1 = _rotl32(x1, d)
            x1 = (x1 ^ x0).astype(np.uint32)
        x0 = (x0 + keys[(r + 1) % 3]).astype(np.uint32)
        x1 = (x1 + keys[(r + 2) % 3] + np.uint32(r + 1)).astype(np.uint32)
    return x0, x1


_NOISE_CACHE = {}


def _noise_for(n):
    if not jax.config.jax_threefry_partitionable:
        return jax.random.uniform(jax.random.key(42), (n, n), jnp.float32) * 0.01
    if n not in _NOISE_CACHE:
        idx = np.arange(n * n, dtype=np.uint64)
        c1 = (idx >> np.uint64(32)).astype(np.uint32)
        c2 = (idx & np.uint64(0xFFFFFFFF)).astype(np.uint32)
        b0, b1 = _threefry2x32_np(np.uint32(0), np.uint32(42), c1, c2)
        bits = (b0 ^ b1).reshape(n, n)
        u = (((bits >> np.uint32(9)) | np.uint32(0x3F800000)).view(np.float32)
             - np.float32(1.0))
        _NOISE_CACHE[n] = np.maximum(np.float32(0), u) * np.float32(0.01)
    return jnp.asarray(_NOISE_CACHE[n])


def _v2_kernel(e2_ref, w2_ref, b2_ref, v2_ref):
    dn = (((1,), (1,)), ((), ()))
    h2 = jax.lax.dot_general(e2_ref[...], w2_ref[...], dn,
                             preferred_element_type=jnp.float32)
    v2_ref[...] = jnp.tanh(h2 + b2_ref[...])


def _topk_mask(ap, noise):
    v = ap + noise
    rows = v.shape[0]
    vb = jax.lax.bitcast_convert_type(v, jnp.int32)
    lo = jnp.full((rows, 1), -1, jnp.int32)
    hi = jax.lax.bitcast_convert_type(jnp.full((rows, 1), 1.02, jnp.float32),
                                      jnp.int32)

    def body(_, carry):
        lo, hi = carry
        mid = lo + ((hi - lo) >> 1)
        cnt = jnp.sum((vb > mid).astype(jnp.int32), axis=1, keepdims=True)
        pred = cnt >= _K
        return jnp.where(pred, mid, lo), jnp.where(pred, hi, mid)

    lo, hi = jax.lax.fori_loop(0, _BS_ITERS, body, (lo, hi))
    gt_hi = vb > hi
    cnt_hi = jnp.sum(gt_hi.astype(jnp.int32), axis=1, keepdims=True)
    need = _K - cnt_hi
    ties = (vb > lo) & (vb <= hi)
    cnt_ties = jnp.sum(ties.astype(jnp.int32), axis=1, keepdims=True)
    col = jax.lax.broadcasted_iota(jnp.int32, v.shape, 1)
    last = v.shape[1] - 1

    def tie_search():
        def body_c(_, carry):
            lo_c, hi_c = carry
            mid = lo_c + ((hi_c - lo_c) >> 1)
            cnt = jnp.sum((ties & (col <= mid)).astype(jnp.int32), axis=1,
                          keepdims=True)
            pred = cnt >= need
            return jnp.where(pred, lo_c, mid), jnp.where(pred, mid, hi_c)

        lo_c = jnp.full((rows, 1), -1, jnp.int32)
        hi_c = jnp.full((rows, 1), last, jnp.int32)
        return jax.lax.fori_loop(0, 11, body_c, (lo_c, hi_c))[1]

    dup = jnp.any(cnt_ties > need)
    hi_c = jax.lax.cond(dup, tie_search,
                        lambda: jnp.full((rows, 1), last, jnp.int32))
    mask = gt_hi | (ties & (col <= hi_c))
    return jnp.where(mask, ap, 0.0)


def _adj_kernel(e1_ref, w1_ref, b1_ref, v2_ref, noise_ref, out_ref, s_ref):
    i = pl.program_id(0)
    p = jax.lax.rem(i, 2)
    dn = (((1,), (1,)), ((), ()))
    h = jnp.tanh(jax.lax.dot_general(e1_ref[...], w1_ref[...], dn,
                                     preferred_element_type=jnp.float32)
                 + b1_ref[...])
    a = jax.nn.sigmoid(jax.lax.dot_general(h, v2_ref[...], dn,
                                           preferred_element_type=jnp.float32))
    s_ref[pl.ds(p * _BLK, _BLK), :] = a

    ap = s_ref[pl.ds((1 - p) * _BLK, _BLK), :]
    out_ref[...] = _topk_mask(ap, noise_ref[...])


def kernel(x, E1, E2, W1, b1, W2, b2):
    n = x.shape[1]
    nblk = n // _BLK
    b1r = b1.reshape(1, n)
    b2r = b2.reshape(1, n)

    v2 = pl.pallas_call(
        _v2_kernel,
        grid=(nblk,),
        in_specs=[
            pl.BlockSpec((_BLK, n), lambda i: (i, 0)),
            pl.BlockSpec((n, n), lambda i: (0, 0)),
            pl.BlockSpec((1, n), lambda i: (0, 0)),
        ],
        out_specs=pl.BlockSpec((_BLK, n), lambda i: (i, 0)),
        out_shape=jax.ShapeDtypeStruct((n, n), jnp.float32),
    )(E2, W2, b2r)

    noise = _noise_for(n)

    out = pl.pallas_call(
        _adj_kernel,
        grid=(nblk + 1,),
        in_specs=[
            pl.BlockSpec((_BLK, n), lambda i: (jnp.minimum(i, nblk - 1), 0)),
            pl.BlockSpec((n, n), lambda i: (0, 0)),
            pl.BlockSpec((1, n), lambda i: (0, 0)),
            pl.BlockSpec((n, n), lambda i: (0, 0)),
            pl.BlockSpec((_BLK, n), lambda i: (jnp.maximum(i - 1, 0), 0)),
        ],
        out_specs=pl.BlockSpec((_BLK, n), lambda i: (jnp.maximum(i - 1, 0), 0)),
        out_shape=jax.ShapeDtypeStruct((n, n), jnp.float32),
        scratch_shapes=[pltpu.VMEM((2 * _BLK, n), jnp.float32)],
    )(E1, W1, b1r, v2, noise)
    return out


# speedup vs baseline: 1.2172x; 1.2172x over previous
import functools

import jax
import jax.numpy as jnp
import numpy as np
from jax.experimental import pallas as pl
from jax.experimental.pallas import tpu as pltpu

_K = 32
_BLK = 256
_BS_ITERS = 31


def _rotl32(x, d):
    x = x.astype(np.uint32)
    return ((x << np.uint32(d)) | (x >> np.uint32(32 - d))).astype(np.uint32)


def _threefry2x32_np(k0, k1, x0, x1):
    r0 = (13, 15, 26, 6)
    r1 = (17, 29, 16, 24)
    keys = (np.uint32(k0), np.uint32(k1),
            np.uint32(np.uint32(k0) ^ np.uint32(k1) ^ np.uint32(0x1BD11BDA)))
    x0 = (x0 + keys[0]).astype(np.uint32)
    x1 = (x1 + keys[1]).astype(np.uint32)
    for r in range(5):
        for d in r0 if r % 2 == 0 else r1:
            x0 = (x0 + x1).astype(np.uint32)
            x1 = _rotl32(x1, d)
            x1 = (x1 ^ x0).astype(np.uint32)
        x0 = (x0 + keys[(r + 1) % 3]).astype(np.uint32)
        x1 = (x1 + keys[(r + 2) % 3] + np.uint32(r + 1)).astype(np.uint32)
    return x0, x1


_NOISE_CACHE = {}


def _noise_for(n):
    if not jax.config.jax_threefry_partitionable:
        return jax.random.uniform(jax.random.key(42), (n, n), jnp.float32) * 0.01
    if n not in _NOISE_CACHE:
        idx = np.arange(n * n, dtype=np.uint64)
        c1 = (idx >> np.uint64(32)).astype(np.uint32)
        c2 = (idx & np.uint64(0xFFFFFFFF)).astype(np.uint32)
        b0, b1 = _threefry2x32_np(np.uint32(0), np.uint32(42), c1, c2)
        bits = (b0 ^ b1).reshape(n, n)
        u = (((bits >> np.uint32(9)) | np.uint32(0x3F800000)).view(np.float32)
             - np.float32(1.0))
        _NOISE_CACHE[n] = np.maximum(np.float32(0), u) * np.float32(0.01)
    return jnp.asarray(_NOISE_CACHE[n])


def _v2_kernel(e2_ref, w2_ref, b2_ref, v2_ref):
    dn = (((1,), (1,)), ((), ()))
    h2 = jax.lax.dot_general(e2_ref[...], w2_ref[...], dn,
                             preferred_element_type=jnp.float32)
    v2_ref[...] = jnp.tanh(h2 + b2_ref[...])


def _topk_mask(ap, noise):
    v = ap + noise
    rows = v.shape[0]
    vb = jax.lax.bitcast_convert_type(v, jnp.int32)
    lo = jnp.full((rows, 1), -1, jnp.int32)
    hi = jax.lax.bitcast_convert_type(jnp.full((rows, 1), 1.02, jnp.float32),
                                      jnp.int32)

    def body(_, carry):
        lo, hi = carry
        mid = lo + ((hi - lo) >> 1)
        cnt = jnp.sum((vb > mid).astype(jnp.int32), axis=1, keepdims=True)
        pred = cnt >= _K
        return jnp.where(pred, mid, lo), jnp.where(pred, hi, mid)

    for it in range(_BS_ITERS):
        lo, hi = body(it, (lo, hi))
    gt_hi = vb > hi
    cnt_hi = jnp.sum(gt_hi.astype(jnp.int32), axis=1, keepdims=True)
    need = _K - cnt_hi
    ties = (vb > lo) & (vb <= hi)
    cnt_ties = jnp.sum(ties.astype(jnp.int32), axis=1, keepdims=True)
    col = jax.lax.broadcasted_iota(jnp.int32, v.shape, 1)
    last = v.shape[1] - 1

    def tie_search():
        def body_c(_, carry):
            lo_c, hi_c = carry
            mid = lo_c + ((hi_c - lo_c) >> 1)
            cnt = jnp.sum((ties & (col <= mid)).astype(jnp.int32), axis=1,
                          keepdims=True)
            pred = cnt >= need
            return jnp.where(pred, lo_c, mid), jnp.where(pred, mid, hi_c)

        lo_c = jnp.full((rows, 1), -1, jnp.int32)
        hi_c = jnp.full((rows, 1), last, jnp.int32)
        nbits = max(1, (v.shape[1]).bit_length() - 1)
        for it in range(nbits):
            lo_c, hi_c = body_c(it, (lo_c, hi_c))
        return hi_c

    dup = jnp.any(cnt_ties > need)
    hi_c = jax.lax.cond(dup, tie_search,
                        lambda: jnp.full((rows, 1), last, jnp.int32))
    mask = gt_hi | (ties & (col <= hi_c))
    return jnp.where(mask, ap, 0.0)


def _adj_kernel(e1_ref, w1_ref, b1_ref, v2_ref, noise_ref, out_ref, s_ref):
    i = pl.program_id(0)
    p = jax.lax.rem(i, 2)
    dn = (((1,), (1,)), ((), ()))
    h = jnp.tanh(jax.lax.dot_general(e1_ref[...], w1_ref[...], dn,
                                     preferred_element_type=jnp.float32)
                 + b1_ref[...])
    a = jax.nn.sigmoid(jax.lax.dot_general(h, v2_ref[...], dn,
                                           preferred_element_type=jnp.float32))
    s_ref[pl.ds(p * _BLK, _BLK), :] = a

    ap = s_ref[pl.ds((1 - p) * _BLK, _BLK), :]
    out_ref[...] = _topk_mask(ap, noise_ref[...])


def kernel(x, E1, E2, W1, b1, W2, b2):
    n = x.shape[1]
    nblk = n // _BLK
    b1r = b1.reshape(1, n)
    b2r = b2.reshape(1, n)

    v2 = pl.pallas_call(
        _v2_kernel,
        grid=(nblk,),
        in_specs=[
            pl.BlockSpec((_BLK, n), lambda i: (i, 0)),
            pl.BlockSpec((n, n), lambda i: (0, 0)),
            pl.BlockSpec((1, n), lambda i: (0, 0)),
        ],
        out_specs=pl.BlockSpec((_BLK, n), lambda i: (i, 0)),
        out_shape=jax.ShapeDtypeStruct((n, n), jnp.float32),
    )(E2, W2, b2r)

    noise = _noise_for(n)

    out = pl.pallas_call(
        _adj_kernel,
        grid=(nblk + 1,),
        in_specs=[
            pl.BlockSpec((_BLK, n), lambda i: (jnp.minimum(i, nblk - 1), 0)),
            pl.BlockSpec((n, n), lambda i: (0, 0)),
            pl.BlockSpec((1, n), lambda i: (0, 0)),
            pl.BlockSpec((n, n), lambda i: (0, 0)),
            pl.BlockSpec((_BLK, n), lambda i: (jnp.maximum(i - 1, 0), 0)),
        ],
        out_specs=pl.BlockSpec((_BLK, n), lambda i: (jnp.maximum(i - 1, 0), 0)),
        out_shape=jax.ShapeDtypeStruct((n, n), jnp.float32),
        scratch_shapes=[pltpu.VMEM((2 * _BLK, n), jnp.float32)],
    )(E1, W1, b1r, v2, noise)
    return out
